# baseline (speedup 1.0000x reference)
import jax
import jax.numpy as jnp
from jax.experimental import pallas as pl

_BR = 2000


def _probe_kernel(x_ref, o_ref):
    s = jnp.sum(x_ref[0:8, :], axis=1, keepdims=True)
    o_ref[...] = jax.lax.broadcast_in_dim(s[0:1], o_ref.shape, (0, 1))


def kernel(X, wK, cK):
    M, K = X.shape
    N = wK.shape[0]
    out = pl.pallas_call(
        _probe_kernel,
        grid=(16000 // _BR,),
        in_specs=[pl.BlockSpec((8, 128), lambda i: (0, 0))],
        out_specs=pl.BlockSpec((_BR, 1024), lambda i: (i, 0)),
        out_shape=jax.ShapeDtypeStruct((16000, 1024), jnp.float32),
    )(X)
    return jnp.reshape(out, (M, N))


# speedup vs baseline: 1.5974x; 1.5974x over previous
import jax
import jax.numpy as jnp
from jax.experimental import pallas as pl

_BM = 2048
_SUB = 512


def _fused_kernel(x_ref, w_ref, c_ref, o_ref):
    i = pl.program_id(0)
    w = w_ref[...]
    nv = c_ref[...] == 0
    for k in range(_BM // _SUB):
        base = i * _BM + k * _SUB
        s = jax.lax.dot_general(
            x_ref[pl.ds(base, _SUB), :], w,
            dimension_numbers=(((1,), (1,)), ((), ())),
            preferred_element_type=jnp.float32,
        )
        mn = jnp.min(s, axis=1, keepdims=True) - 1.0
        o_ref[pl.ds(k * _SUB, _SUB), :] = jnp.where(nv, mn, s)


def kernel(X, wK, cK):
    M, K = X.shape
    N = wK.shape[0]
    c2d = cK.reshape(1, N)
    grid = (M // _BM,)
    return pl.pallas_call(
        _fused_kernel,
        grid=grid,
        in_specs=[
            pl.BlockSpec((M, K), lambda i: (0, 0)),
            pl.BlockSpec((N, K), lambda i: (0, 0)),
            pl.BlockSpec((1, N), lambda i: (0, 0)),
        ],
        out_specs=pl.BlockSpec((_BM, N), lambda i: (i, 0)),
        out_shape=jax.ShapeDtypeStruct((M, N), jnp.float32),
    )(X, wK, c2d)


# speedup vs baseline: 1.6347x; 1.0234x over previous
import jax
import jax.numpy as jnp
from jax.experimental import pallas as pl

_BM = 2048
_SUB = 512


def _fused_kernel(x_ref, w_ref, c_ref, o_ref):
    w = w_ref[...]
    nv = c_ref[...] == 0
    for base in range(0, _BM, _SUB):
        s = jax.lax.dot_general(
            x_ref[base:base + _SUB, :], w,
            dimension_numbers=(((1,), (1,)), ((), ())),
            preferred_element_type=jnp.float32,
        )
        mn = jnp.min(s, axis=1, keepdims=True) - 1.0
        o_ref[base:base + _SUB, :] = jnp.where(nv, mn, s)


def kernel(X, wK, cK):
    M, K = X.shape
    N = wK.shape[0]
    c2d = cK.reshape(1, N)
    grid = (M // _BM,)
    return pl.pallas_call(
        _fused_kernel,
        grid=grid,
        in_specs=[
            pl.BlockSpec((_BM, K), lambda i: (i, 0)),
            pl.BlockSpec((N, K), lambda i: (0, 0)),
            pl.BlockSpec((1, N), lambda i: (0, 0)),
        ],
        out_specs=pl.BlockSpec((_BM, N), lambda i: (i, 0)),
        out_shape=jax.ShapeDtypeStruct((M, N), jnp.float32),
    )(X, wK, c2d)
